# initial kernel scaffold (unmeasured)
import functools

import jax
import jax.numpy as jnp
from jax import lax
from jax.experimental import pallas as pl
from jax.experimental.pallas import tpu as pltpu

N_DEV = 8


def kernel(x, router_W, route_idx, expert_W):
    T, D = x.shape
    E_LOCAL, _, H = expert_W.shape
    N_EXP = router_W.shape[1]

    def body(x_ref, rw_ref, idx_ref, ew_ref, out_ref,
             comm_ref, send_sems, recv_sems, credit_sem):
        my = lax.axis_index("i")
        left = lax.rem(my - 1 + N_DEV, N_DEV)
        right = lax.rem(my + 1, N_DEV)

        barrier = pltpu.get_barrier_semaphore()
        for nbr in (left, right):
            pl.semaphore_signal(barrier, inc=1, device_id=(nbr,),
                                device_id_type=pl.DeviceIdType.MESH)
        pl.semaphore_wait(barrier, 2)

        xv = x_ref[...]
        scores = jnp.dot(xv, rw_ref[...], preferred_element_type=jnp.float32)
        e0 = idx_ref[:, 0:1]
        e1 = idx_ref[:, 1:2]
        eids = lax.broadcasted_iota(jnp.int32, (T, N_EXP), 1)
        s0 = jnp.sum(jnp.where(eids == e0, scores, 0.0), axis=1)
        s1 = jnp.sum(jnp.where(eids == e1, scores, 0.0), axis=1)
        w0 = jax.nn.sigmoid(s0 - s1)
        w1 = 1.0 - w0

        def contribution(block, origin):
            ge = origin * E_LOCAL + lax.broadcasted_iota(
                jnp.int32, (1, E_LOCAL), 1)
            coeff = (w0[:, None] * (e0 == ge).astype(jnp.float32)
                     + w1[:, None] * (e1 == ge).astype(jnp.float32))
            xs = (xv[:, None, :] * coeff[:, :, None]).reshape(T, E_LOCAL * D)
            return jnp.dot(xs, block.reshape(E_LOCAL * D, H),
                           preferred_element_type=jnp.float32)

        out_ref[...] = contribution(ew_ref[...], my)

        for h in range(1, N_DEV):
            send_slot = (h - 1) % 2
            recv_slot = h % 2
            if h >= 2:
                pl.semaphore_wait(credit_sem, 1)
            src = ew_ref if h == 1 else comm_ref.at[send_slot]
            rdma = pltpu.make_async_remote_copy(
                src_ref=src,
                dst_ref=comm_ref.at[recv_slot],
                send_sem=send_sems.at[send_slot],
                recv_sem=recv_sems.at[recv_slot],
                device_id=(right,),
                device_id_type=pl.DeviceIdType.MESH,
            )
            rdma.start()
            rdma.wait()
            if h <= N_DEV - 2:
                pl.semaphore_signal(credit_sem, inc=1, device_id=(left,),
                                    device_id_type=pl.DeviceIdType.MESH)
            origin = lax.rem(my - h + N_DEV, N_DEV)
            out_ref[...] += contribution(comm_ref[recv_slot], origin)

        @functools.partial(pl.run_scoped,
                           exit_sem=pltpu.SemaphoreType.REGULAR)
        def _(exit_sem):
            for nbr in (left, right):
                pl.semaphore_signal(exit_sem, inc=1, device_id=(nbr,),
                                    device_id_type=pl.DeviceIdType.MESH)
            pl.semaphore_wait(exit_sem, 2)

    return pl.pallas_call(
        body,
        out_shape=jax.ShapeDtypeStruct((T, H), jnp.float32),
        in_specs=[pl.BlockSpec(memory_space=pltpu.VMEM)] * 4,
        out_specs=pl.BlockSpec(memory_space=pltpu.VMEM),
        scratch_shapes=[
            pltpu.VMEM((2, E_LOCAL, D, H), jnp.float32),
            pltpu.SemaphoreType.DMA((2,)),
            pltpu.SemaphoreType.DMA((2,)),
            pltpu.SemaphoreType.REGULAR,
        ],
        compiler_params=pltpu.CompilerParams(collective_id=0),
    )(x, router_W, route_idx, expert_W)


# baseline (device time: 716969 ns/iter reference)
import functools

import jax
import jax.numpy as jnp
from jax import lax
from jax.experimental import pallas as pl
from jax.experimental.pallas import tpu as pltpu

N_DEV = 8


def kernel(x, router_W, route_idx, expert_W):
    T, D = x.shape
    E_LOCAL, _, H = expert_W.shape
    N_EXP = router_W.shape[1]

    def body(x_ref, rw_ref, idx_ref, ew_ref, out_ref,
             comm_ref, send_sems, recv_sems, credit_sem):
        my = lax.axis_index("i")
        left = lax.rem(my - 1 + N_DEV, N_DEV)
        right = lax.rem(my + 1, N_DEV)

        barrier = pltpu.get_barrier_semaphore()
        for nbr in (left, right):
            pl.semaphore_signal(barrier, inc=1, device_id=(nbr,),
                                device_id_type=pl.DeviceIdType.MESH)
        pl.semaphore_wait(barrier, 2)

        xv = x_ref[...]
        scores = jnp.dot(xv, rw_ref[...], preferred_element_type=jnp.float32)
        e0 = idx_ref[:, 0:1]
        e1 = idx_ref[:, 1:2]
        eids = lax.broadcasted_iota(jnp.int32, (T, N_EXP), 1)
        s0 = jnp.sum(jnp.where(eids == e0, scores, 0.0), axis=1)
        s1 = jnp.sum(jnp.where(eids == e1, scores, 0.0), axis=1)
        w0 = jax.nn.sigmoid(s0 - s1)
        w1 = 1.0 - w0

        def contribution(block, origin):
            ge = origin * E_LOCAL + lax.broadcasted_iota(
                jnp.int32, (1, E_LOCAL), 1)
            coeff = (w0[:, None] * (e0 == ge).astype(jnp.float32)
                     + w1[:, None] * (e1 == ge).astype(jnp.float32))
            xs = (xv[:, None, :] * coeff[:, :, None]).reshape(T, E_LOCAL * D)
            return jnp.dot(xs, block.reshape(E_LOCAL * D, H),
                           preferred_element_type=jnp.float32)

        out_ref[...] = contribution(ew_ref[...], my)

        for h in range(1, N_DEV):
            send_slot = (h - 1) % 2
            recv_slot = h % 2
            if h >= 2:
                pl.semaphore_wait(credit_sem, 1)
            src = ew_ref if h == 1 else comm_ref.at[send_slot]
            rdma = pltpu.make_async_remote_copy(
                src_ref=src,
                dst_ref=comm_ref.at[recv_slot],
                send_sem=send_sems.at[send_slot],
                recv_sem=recv_sems.at[recv_slot],
                device_id=(right,),
                device_id_type=pl.DeviceIdType.MESH,
            )
            rdma.start()
            rdma.wait()
            if h <= N_DEV - 2:
                pl.semaphore_signal(credit_sem, inc=1, device_id=(left,),
                                    device_id_type=pl.DeviceIdType.MESH)
            origin = lax.rem(my - h + N_DEV, N_DEV)
            out_ref[...] += contribution(comm_ref[recv_slot], origin)

        @functools.partial(pl.run_scoped,
                           exit_sem=pltpu.SemaphoreType.REGULAR)
        def _(exit_sem):
            for nbr in (left, right):
                pl.semaphore_signal(exit_sem, inc=1, device_id=(nbr,),
                                    device_id_type=pl.DeviceIdType.MESH)
            pl.semaphore_wait(exit_sem, 2)

    return pl.pallas_call(
        body,
        out_shape=jax.ShapeDtypeStruct((T, H), jnp.float32),
        in_specs=[pl.BlockSpec(memory_space=pltpu.VMEM)] * 4,
        out_specs=pl.BlockSpec(memory_space=pltpu.VMEM),
        scratch_shapes=[
            pltpu.VMEM((2, E_LOCAL, D, H), jnp.float32),
            pltpu.SemaphoreType.DMA((2,)),
            pltpu.SemaphoreType.DMA((2,)),
            pltpu.SemaphoreType.REGULAR,
        ],
        compiler_params=pltpu.CompilerParams(
            collective_id=0,
            vmem_limit_bytes=100 * 1024 * 1024,
        ),
    )(x, router_W, route_idx, expert_W)


# device time: 364428 ns/iter; 1.9674x vs baseline; 1.9674x over previous
import functools

import jax
import jax.numpy as jnp
from jax import lax
from jax.experimental import pallas as pl
from jax.experimental.pallas import tpu as pltpu

N_DEV = 8


def kernel(x, router_W, route_idx, expert_W):
    T, D = x.shape
    E_LOCAL, _, H = expert_W.shape
    E_HALF = E_LOCAL // 2
    N_EXP = router_W.shape[1]

    def body(x_ref, rw_ref, idx_ref, ew_ref, out_ref,
             commR_ref, commL_ref,
             sendR_sems, recvR_sems, sendL_sems, recvL_sems,
             creditR_sem, creditL_sem):
        my = lax.axis_index("i")
        left = lax.rem(my - 1 + N_DEV, N_DEV)
        right = lax.rem(my + 1, N_DEV)

        barrier = pltpu.get_barrier_semaphore()
        for nbr in (left, right):
            pl.semaphore_signal(barrier, inc=1, device_id=(nbr,),
                                device_id_type=pl.DeviceIdType.MESH)
        pl.semaphore_wait(barrier, 2)

        xv = x_ref[...]
        scores = jnp.dot(xv, rw_ref[...], preferred_element_type=jnp.float32)
        e0 = idx_ref[:, 0:1]
        e1 = idx_ref[:, 1:2]
        eids = lax.broadcasted_iota(jnp.int32, (T, N_EXP), 1)
        s0 = jnp.sum(jnp.where(eids == e0, scores, 0.0), axis=1)
        s1 = jnp.sum(jnp.where(eids == e1, scores, 0.0), axis=1)
        w0 = jax.nn.sigmoid(s0 - s1)
        w1 = 1.0 - w0

        def contribution(block, base, n_exp):
            ge = base + lax.broadcasted_iota(jnp.int32, (1, n_exp), 1)
            coeff = (w0[:, None] * (e0 == ge).astype(jnp.float32)
                     + w1[:, None] * (e1 == ge).astype(jnp.float32))
            xs = (xv[:, None, :] * coeff[:, :, None]).reshape(T, n_exp * D)
            return jnp.dot(xs, block.reshape(n_exp * D, H),
                           preferred_element_type=jnp.float32)

        commR_ref[0, ...] = ew_ref[0:E_HALF, ...]
        commL_ref[0, ...] = ew_ref[E_HALF:E_LOCAL, ...]
        for h in range(1, N_DEV):
            send_slot = (h - 1) % 2
            recv_slot = h % 2
            if h >= 2:
                pl.semaphore_wait(creditR_sem, 1)
                pl.semaphore_wait(creditL_sem, 1)
            rdmaR = pltpu.make_async_remote_copy(
                src_ref=commR_ref.at[send_slot],
                dst_ref=commR_ref.at[recv_slot],
                send_sem=sendR_sems.at[send_slot],
                recv_sem=recvR_sems.at[recv_slot],
                device_id=(right,),
                device_id_type=pl.DeviceIdType.MESH,
            )
            rdmaL = pltpu.make_async_remote_copy(
                src_ref=commL_ref.at[send_slot],
                dst_ref=commL_ref.at[recv_slot],
                send_sem=sendL_sems.at[send_slot],
                recv_sem=recvL_sems.at[recv_slot],
                device_id=(left,),
                device_id_type=pl.DeviceIdType.MESH,
            )
            rdmaR.start()
            rdmaL.start()
            if h == 1:
                out_ref[...] = contribution(ew_ref[...], my * E_LOCAL,
                                            E_LOCAL)
            else:
                oR = lax.rem(my - (h - 1) + N_DEV, N_DEV)
                oL = lax.rem(my + (h - 1), N_DEV)
                out_ref[...] += contribution(
                    commR_ref[send_slot], oR * E_LOCAL, E_HALF)
                out_ref[...] += contribution(
                    commL_ref[send_slot], oL * E_LOCAL + E_HALF, E_HALF)
            rdmaR.wait()
            rdmaL.wait()
            if h <= N_DEV - 2:
                pl.semaphore_signal(creditR_sem, inc=1, device_id=(left,),
                                    device_id_type=pl.DeviceIdType.MESH)
                pl.semaphore_signal(creditL_sem, inc=1, device_id=(right,),
                                    device_id_type=pl.DeviceIdType.MESH)
        last = (N_DEV - 1) % 2
        oR = lax.rem(my - (N_DEV - 1) + N_DEV, N_DEV)
        oL = lax.rem(my + (N_DEV - 1), N_DEV)
        out_ref[...] += contribution(commR_ref[last], oR * E_LOCAL, E_HALF)
        out_ref[...] += contribution(commL_ref[last],
                                     oL * E_LOCAL + E_HALF, E_HALF)

        @functools.partial(pl.run_scoped,
                           exit_sem=pltpu.SemaphoreType.REGULAR)
        def _(exit_sem):
            for nbr in (left, right):
                pl.semaphore_signal(exit_sem, inc=1, device_id=(nbr,),
                                    device_id_type=pl.DeviceIdType.MESH)
            pl.semaphore_wait(exit_sem, 2)

    return pl.pallas_call(
        body,
        out_shape=jax.ShapeDtypeStruct((T, H), jnp.float32),
        in_specs=[pl.BlockSpec(memory_space=pltpu.VMEM)] * 4,
        out_specs=pl.BlockSpec(memory_space=pltpu.VMEM),
        scratch_shapes=[
            pltpu.VMEM((2, E_HALF, D, H), jnp.float32),
            pltpu.VMEM((2, E_HALF, D, H), jnp.float32),
            pltpu.SemaphoreType.DMA((2,)),
            pltpu.SemaphoreType.DMA((2,)),
            pltpu.SemaphoreType.DMA((2,)),
            pltpu.SemaphoreType.DMA((2,)),
            pltpu.SemaphoreType.REGULAR,
            pltpu.SemaphoreType.REGULAR,
        ],
        compiler_params=pltpu.CompilerParams(
            collective_id=0,
            vmem_limit_bytes=100 * 1024 * 1024,
        ),
    )(x, router_W, route_idx, expert_W)


# device time: 207408 ns/iter; 3.4568x vs baseline; 1.7571x over previous
import functools

import jax
import jax.numpy as jnp
from jax import lax
from jax.experimental import pallas as pl
from jax.experimental.pallas import tpu as pltpu

N_DEV = 8


def kernel(x, router_W, route_idx, expert_W):
    T, D = x.shape
    E_LOCAL, _, H = expert_W.shape
    E_HALF = E_LOCAL // 2
    N_EXP = router_W.shape[1]

    def body(x_ref, rw_ref, idx_ref, ew_ref, out_ref,
             commR_ref, commL_ref,
             sendR_sems, recvR_sems, sendL_sems, recvL_sems,
             creditR_sem, creditL_sem):
        my = lax.axis_index("i")
        left = lax.rem(my - 1 + N_DEV, N_DEV)
        right = lax.rem(my + 1, N_DEV)

        barrier = pltpu.get_barrier_semaphore()
        for nbr in (left, right):
            pl.semaphore_signal(barrier, inc=1, device_id=(nbr,),
                                device_id_type=pl.DeviceIdType.MESH)
        pl.semaphore_wait(barrier, 2)

        xv = x_ref[...]
        scores = jnp.dot(xv, rw_ref[...], preferred_element_type=jnp.float32)
        e0 = idx_ref[:, 0:1]
        e1 = idx_ref[:, 1:2]
        eids = lax.broadcasted_iota(jnp.int32, (T, N_EXP), 1)
        s0 = jnp.sum(jnp.where(eids == e0, scores, 0.0), axis=1)
        s1 = jnp.sum(jnp.where(eids == e1, scores, 0.0), axis=1)
        w0 = jax.nn.sigmoid(s0 - s1)
        w1 = 1.0 - w0

        def contribution(block, base, n_exp):
            ge = base + lax.broadcasted_iota(jnp.int32, (1, n_exp), 1)
            coeff = (w0[:, None] * (e0 == ge).astype(jnp.float32)
                     + w1[:, None] * (e1 == ge).astype(jnp.float32))
            xs = (xv[:, None, :] * coeff[:, :, None]).reshape(
                T, n_exp * D).astype(jnp.bfloat16)
            blk = block.reshape(n_exp * D, H).astype(jnp.bfloat16)
            return jnp.dot(xs, blk, preferred_element_type=jnp.float32)

        commR_ref[0, ...] = ew_ref[0:E_HALF, ...].astype(jnp.bfloat16)
        commL_ref[0, ...] = ew_ref[E_HALF:E_LOCAL, ...].astype(jnp.bfloat16)
        for h in range(1, N_DEV):
            send_slot = (h - 1) % 2
            recv_slot = h % 2
            if h >= 2:
                pl.semaphore_wait(creditR_sem, 1)
                pl.semaphore_wait(creditL_sem, 1)
            rdmaR = pltpu.make_async_remote_copy(
                src_ref=commR_ref.at[send_slot],
                dst_ref=commR_ref.at[recv_slot],
                send_sem=sendR_sems.at[send_slot],
                recv_sem=recvR_sems.at[recv_slot],
                device_id=(right,),
                device_id_type=pl.DeviceIdType.MESH,
            )
            rdmaL = pltpu.make_async_remote_copy(
                src_ref=commL_ref.at[send_slot],
                dst_ref=commL_ref.at[recv_slot],
                send_sem=sendL_sems.at[send_slot],
                recv_sem=recvL_sems.at[recv_slot],
                device_id=(left,),
                device_id_type=pl.DeviceIdType.MESH,
            )
            rdmaR.start()
            rdmaL.start()
            if h == 1:
                out_ref[...] = contribution(ew_ref[...], my * E_LOCAL,
                                            E_LOCAL)
            else:
                oR = lax.rem(my - (h - 1) + N_DEV, N_DEV)
                oL = lax.rem(my + (h - 1), N_DEV)
                out_ref[...] += contribution(
                    commR_ref[send_slot], oR * E_LOCAL, E_HALF)
                out_ref[...] += contribution(
                    commL_ref[send_slot], oL * E_LOCAL + E_HALF, E_HALF)
            rdmaR.wait()
            rdmaL.wait()
            if h <= N_DEV - 2:
                pl.semaphore_signal(creditR_sem, inc=1, device_id=(left,),
                                    device_id_type=pl.DeviceIdType.MESH)
                pl.semaphore_signal(creditL_sem, inc=1, device_id=(right,),
                                    device_id_type=pl.DeviceIdType.MESH)
        last = (N_DEV - 1) % 2
        oR = lax.rem(my - (N_DEV - 1) + N_DEV, N_DEV)
        oL = lax.rem(my + (N_DEV - 1), N_DEV)
        out_ref[...] += contribution(commR_ref[last], oR * E_LOCAL, E_HALF)
        out_ref[...] += contribution(commL_ref[last],
                                     oL * E_LOCAL + E_HALF, E_HALF)

        @functools.partial(pl.run_scoped,
                           exit_sem=pltpu.SemaphoreType.REGULAR)
        def _(exit_sem):
            for nbr in (left, right):
                pl.semaphore_signal(exit_sem, inc=1, device_id=(nbr,),
                                    device_id_type=pl.DeviceIdType.MESH)
            pl.semaphore_wait(exit_sem, 2)

    return pl.pallas_call(
        body,
        out_shape=jax.ShapeDtypeStruct((T, H), jnp.float32),
        in_specs=[pl.BlockSpec(memory_space=pltpu.VMEM)] * 4,
        out_specs=pl.BlockSpec(memory_space=pltpu.VMEM),
        scratch_shapes=[
            pltpu.VMEM((2, E_HALF, D, H), jnp.bfloat16),
            pltpu.VMEM((2, E_HALF, D, H), jnp.bfloat16),
            pltpu.SemaphoreType.DMA((2,)),
            pltpu.SemaphoreType.DMA((2,)),
            pltpu.SemaphoreType.DMA((2,)),
            pltpu.SemaphoreType.DMA((2,)),
            pltpu.SemaphoreType.REGULAR,
            pltpu.SemaphoreType.REGULAR,
        ],
        compiler_params=pltpu.CompilerParams(
            collective_id=0,
            vmem_limit_bytes=100 * 1024 * 1024,
        ),
    )(x, router_W, route_idx, expert_W)
